# baseline (speedup 1.0000x reference)
import functools

import jax
import jax.numpy as jnp
from jax import lax
from jax.experimental import pallas as pl
from jax.experimental.pallas import tpu as pltpu
from jax.experimental.pallas import tpu_sc as plsc

DIM = 1024
NUM_EXPERTS = 8
FFN = 2048
OUT = 1024
B = 2
N = 2048
TOPK = 256


def _sortable_keys(v):
    b = lax.bitcast_convert_type(v, jnp.int32)
    return jnp.where(b < 0, jnp.bitwise_xor(jnp.invert(b), -2**31), b)


def _cumsum_lanes(x):
    s = x
    k = 1
    while k < N:
        s = s + jnp.concatenate(
            [jnp.zeros((s.shape[0], k), s.dtype), s[:, :N - k]], axis=1)
        k *= 2
    return s


def _gate_kernel(x_ref, wg_ref, bgr_ref, bgc_ref,
                 xg_ref, srank_ref, wm_ref, idx_ref):
    xb = x_ref[0]
    logits = lax.dot_general(
        xb, wg_ref[...], (((1,), (0,)), ((), ())),
        precision=lax.Precision.DEFAULT,
        preferred_element_type=jnp.float32) + bgr_ref[...]
    xg_ref[0] = logits
    lt = lax.dot_general(
        wg_ref[...], xb, (((0,), (1,)), ((), ())),
        precision=lax.Precision.DEFAULT,
        preferred_element_type=jnp.float32) + bgc_ref[...]

    keys = _sortable_keys(lt)
    lo = jnp.full((NUM_EXPERTS, 1), -2**31, jnp.int32)
    hi = jnp.full((NUM_EXPERTS, 1), 2**31 - 1, jnp.int32)
    for _ in range(33):
        mid = (lo >> 1) + (hi >> 1) + (jnp.bitwise_or(lo, hi) & 1)
        cnt = jnp.sum((keys >= mid).astype(jnp.int32), axis=1, keepdims=True)
        ok = cnt >= TOPK
        lo = jnp.where(ok, mid, lo)
        hi = jnp.where(ok, hi, mid - 1)
    thr = lo

    gt = keys > thr
    eq = keys == thr
    n_gt = jnp.sum(gt.astype(jnp.int32), axis=1, keepdims=True)
    needed = (TOPK - n_gt).astype(jnp.float32)
    cum_eq = _cumsum_lanes(eq.astype(jnp.float32))
    sel = jnp.logical_or(gt, jnp.logical_and(eq, cum_eq <= needed))
    rank = _cumsum_lanes(sel.astype(jnp.float32)) - 1.0
    srm = jnp.where(sel, rank, -1.0)
    srank_ref[0] = srm.reshape(NUM_EXPERTS, 1, N)

    ni = lax.broadcasted_iota(jnp.int32, (2, N), 1)
    qr = jnp.concatenate([
        lax.slice_in_dim(ni >> 4, 0, 1, axis=0),
        lax.slice_in_dim(ni & 15, 1, 2, axis=0)], axis=0).astype(jnp.bfloat16)
    iota_k = lax.broadcasted_iota(jnp.int32, (TOPK, N), 0).astype(jnp.float32)
    bb = pl.program_id(0)
    rows = []
    for e in range(NUM_EXPERTS):
        sre = lax.slice_in_dim(srm, e, e + 1, axis=0)
        pmat = (iota_k == sre).astype(jnp.bfloat16)
        qri = lax.dot_general(qr, pmat, (((1,), (1,)), ((), ())),
                              preferred_element_type=jnp.float32)
        rows.append(16.0 * lax.slice_in_dim(qri, 0, 1, axis=0)
                    + lax.slice_in_dim(qri, 1, 2, axis=0))
    idxf = jnp.concatenate(rows, axis=0)
    idx_ref[0] = idxf.astype(jnp.int32) + bb * N

    m = jnp.max(lt, axis=0, keepdims=True)
    p = jnp.exp(lt - m)
    w = p / jnp.sum(p, axis=0, keepdims=True)
    wm_ref[0] = jnp.where(sel, w, 0.0).reshape(NUM_EXPERTS, 1, N)


def _route_gather_kernel(idx_hbm, x_hbm, xe_hbm, idx_v, rows_v, sem):
    wid = lax.axis_index("s") * 2 + lax.axis_index("c")
    pair = wid // 2
    h = wid % 2
    b = pair % B
    e = pair // B
    row = b * NUM_EXPERTS + e
    for c in range(2):
        pltpu.sync_copy(idx_hbm.at[row, pl.ds(h * 128 + c * 64, 64)], idx_v)
        pltpu.async_copy(x_hbm.at[idx_v], rows_v, sem).wait()
        pltpu.sync_copy(rows_v, xe_hbm.at[wid, pl.ds(c * 64, 64)])


def _ffnc_kernel(xe_ref, sr_ref, wm_ref, w1_ref, b1_ref, g_ref, bt_ref,
                 w2_any, b2_ref, out_ref, w2_buf, sem):
    e = pl.program_id(0)
    cp = pltpu.make_async_copy(w2_any.at[e], w2_buf, sem)
    cp.start()
    xe = xe_ref[0].reshape(B * TOPK, DIM).astype(jnp.bfloat16)
    h = lax.dot_general(
        xe, w1_ref[0].astype(jnp.bfloat16), (((1,), (0,)), ((), ())),
        preferred_element_type=jnp.float32) + b1_ref[0]
    h = 0.5 * h * (1.0 + lax.erf(h * 0.7071067811865476))
    mu = jnp.mean(h, axis=1, keepdims=True)
    var = jnp.mean((h - mu) ** 2, axis=1, keepdims=True)
    hn = (h - mu) * lax.rsqrt(var + 1e-5)
    y = (hn * g_ref[0] + bt_ref[0]).astype(jnp.bfloat16)
    cp.wait()
    y2 = lax.dot_general(
        y, w2_buf[...].astype(jnp.bfloat16), (((1,), (0,)), ((), ())),
        preferred_element_type=jnp.float32) + b2_ref[0]
    iota_k = lax.broadcasted_iota(jnp.int32, (TOPK, N), 0).astype(jnp.float32)
    for b in range(B):
        sre = sr_ref[b, 0]
        wme = wm_ref[b, 0]
        pw = jnp.where(iota_k == sre, wme, 0.0).astype(jnp.bfloat16)
        y2b = lax.slice_in_dim(y2, b * TOPK, (b + 1) * TOPK, axis=0)
        contrib = lax.dot_general(
            pw, y2b.astype(jnp.bfloat16), (((0,), (0,)), ((), ())),
            preferred_element_type=jnp.float32)

        @pl.when(e == 0)
        def _():
            out_ref[b] = contrib

        @pl.when(e > 0)
        def _():
            out_ref[b] = out_ref[b] + contrib


@jax.jit
def kernel(x, Wg, bg, W1, b1, gamma, beta, W2, b2):
    f32 = jnp.float32
    bf16 = jnp.bfloat16
    gate = pl.pallas_call(
        _gate_kernel,
        grid=(B,),
        in_specs=[
            pl.BlockSpec((1, N, DIM), lambda b: (b, 0, 0)),
            pl.BlockSpec((DIM, NUM_EXPERTS), lambda b: (0, 0)),
            pl.BlockSpec((1, NUM_EXPERTS), lambda b: (0, 0)),
            pl.BlockSpec((NUM_EXPERTS, 1), lambda b: (0, 0)),
        ],
        out_specs=[
            pl.BlockSpec((1, N, NUM_EXPERTS), lambda b: (b, 0, 0)),
            pl.BlockSpec((1, NUM_EXPERTS, 1, N), lambda b: (b, 0, 0, 0)),
            pl.BlockSpec((1, NUM_EXPERTS, 1, N), lambda b: (b, 0, 0, 0)),
            pl.BlockSpec((1, NUM_EXPERTS, TOPK), lambda b: (b, 0, 0)),
        ],
        out_shape=[
            jax.ShapeDtypeStruct((B, N, NUM_EXPERTS), f32),
            jax.ShapeDtypeStruct((B, NUM_EXPERTS, 1, N), f32),
            jax.ShapeDtypeStruct((B, NUM_EXPERTS, 1, N), f32),
            jax.ShapeDtypeStruct((B, NUM_EXPERTS, TOPK), jnp.int32),
        ],
    )
    x_gated, srank, wm, idxg = gate(
        x, Wg, bg.reshape(1, NUM_EXPERTS), bg.reshape(NUM_EXPERTS, 1))

    route_gather = functools.partial(
        pl.kernel,
        out_type=jax.ShapeDtypeStruct((2 * NUM_EXPERTS * B, 128, DIM), f32),
        mesh=plsc.VectorSubcoreMesh(core_axis_name="c", subcore_axis_name="s"),
        scratch_types=[
            pltpu.VMEM((64,), jnp.int32),
            pltpu.VMEM((64, DIM), jnp.float32),
            pltpu.SemaphoreType.DMA,
        ],
    )(_route_gather_kernel)
    xe32 = route_gather(idxg.reshape(B * NUM_EXPERTS, TOPK),
                        x.reshape(B * N, DIM))
    xe = xe32.reshape(NUM_EXPERTS, B, TOPK, DIM)

    moe = pl.pallas_call(
        _ffnc_kernel,
        grid=(NUM_EXPERTS,),
        in_specs=[
            pl.BlockSpec((1, B, TOPK, DIM), lambda e: (e, 0, 0, 0)),
            pl.BlockSpec((B, 1, 1, N), lambda e: (0, e, 0, 0)),
            pl.BlockSpec((B, 1, 1, N), lambda e: (0, e, 0, 0)),
            pl.BlockSpec((1, DIM, FFN), lambda e: (e, 0, 0)),
            pl.BlockSpec((1, 1, FFN), lambda e: (e, 0, 0)),
            pl.BlockSpec((1, 1, FFN), lambda e: (e, 0, 0)),
            pl.BlockSpec((1, 1, FFN), lambda e: (e, 0, 0)),
            pl.BlockSpec(memory_space=pl.ANY),
            pl.BlockSpec((1, 1, OUT), lambda e: (e, 0, 0)),
        ],
        out_specs=pl.BlockSpec((B, N, OUT), lambda e: (0, 0, 0)),
        out_shape=jax.ShapeDtypeStruct((B, N, OUT), f32),
        scratch_shapes=[
            pltpu.VMEM((FFN, OUT), f32),
            pltpu.SemaphoreType.DMA,
        ],
    )(xe, srank, wm, W1, b1.reshape(NUM_EXPERTS, 1, FFN),
      gamma.reshape(NUM_EXPERTS, 1, FFN), beta.reshape(NUM_EXPERTS, 1, FFN),
      W2, b2.reshape(NUM_EXPERTS, 1, OUT))
    return (moe, x_gated)


# speedup vs baseline: 1.1420x; 1.1420x over previous
import functools

import jax
import jax.numpy as jnp
from jax import lax
from jax.experimental import pallas as pl
from jax.experimental.pallas import tpu as pltpu

DIM = 1024
NUM_EXPERTS = 8
FFN = 2048
OUT = 1024
B = 2
N = 2048
TOPK = 256

def _sortable_keys(v):
    b = lax.bitcast_convert_type(v, jnp.int32)
    return jnp.where(b < 0, jnp.bitwise_xor(jnp.invert(b), -2**31), b)


def _cumsum_lanes(x):
    s = x
    k = 1
    while k < N:
        s = s + jnp.concatenate(
            [jnp.zeros((s.shape[0], k), s.dtype), s[:, :N - k]], axis=1)
        k *= 2
    return s


def _gate_kernel(x_ref, wg_ref, bgr_ref, bgc_ref,
                 xg_ref, srank_ref, wm_ref, xbf_ref):
    xb = x_ref[0]
    logits = lax.dot_general(
        xb, wg_ref[...], (((1,), (0,)), ((), ())),
        precision=lax.Precision.DEFAULT,
        preferred_element_type=jnp.float32) + bgr_ref[...]
    xg_ref[0] = logits
    lt = lax.dot_general(
        wg_ref[...], xb, (((0,), (1,)), ((), ())),
        precision=lax.Precision.DEFAULT,
        preferred_element_type=jnp.float32) + bgc_ref[...]

    keys = _sortable_keys(lt)
    lo = jnp.full((NUM_EXPERTS, 1), -2**31, jnp.int32)
    hi = jnp.full((NUM_EXPERTS, 1), 2**31 - 1, jnp.int32)

    for _ in range(33):
        mid = (lo >> 1) + (hi >> 1) + (jnp.bitwise_or(lo, hi) & 1)
        cnt = jnp.sum((keys >= mid).astype(jnp.int32), axis=1, keepdims=True)
        ok = cnt >= TOPK
        lo = jnp.where(ok, mid, lo)
        hi = jnp.where(ok, hi, mid - 1)
    thr = lo

    gt = keys > thr
    eq = keys == thr
    n_gt = jnp.sum(gt.astype(jnp.int32), axis=1, keepdims=True)
    needed = (TOPK - n_gt).astype(jnp.float32)
    cum_eq = _cumsum_lanes(eq.astype(jnp.float32))
    sel = jnp.logical_or(gt, jnp.logical_and(eq, cum_eq <= needed))
    rank = _cumsum_lanes(sel.astype(jnp.float32)) - 1.0
    srank_ref[0] = jnp.where(sel, rank, -1.0).reshape(NUM_EXPERTS, 1, N)

    m = jnp.max(lt, axis=0, keepdims=True)
    p = jnp.exp(lt - m)
    w = p / jnp.sum(p, axis=0, keepdims=True)
    wm_ref[0] = jnp.where(sel, w, 0.0).reshape(NUM_EXPERTS, 1, N)
    xbf_ref[0] = xb.astype(jnp.bfloat16)


def _ffnc_kernel(xbf_ref, sr_ref, wm_ref, w1_ref, b1_ref, g_ref, bt_ref,
                 w2_any, b2_ref, out_ref, w2_buf, sem):
    e = pl.program_id(0)
    cp = pltpu.make_async_copy(w2_any.at[e], w2_buf, sem)
    cp.start()
    iota_k = lax.broadcasted_iota(jnp.int32, (TOPK, N), 0).astype(jnp.float32)
    xes = []
    for b in range(B):
        sr = sr_ref[b, 0]
        pmat = (iota_k == sr).astype(jnp.bfloat16)
        xes.append(lax.dot_general(
            pmat, xbf_ref[b], (((1,), (0,)), ((), ())),
            preferred_element_type=jnp.float32))
    xe = jnp.concatenate(xes, axis=0).astype(jnp.bfloat16)
    h = lax.dot_general(
        xe, w1_ref[0].astype(jnp.bfloat16), (((1,), (0,)), ((), ())),
        preferred_element_type=jnp.float32) + b1_ref[0]
    h = 0.5 * h * (1.0 + lax.erf(h * 0.7071067811865476))
    mu = jnp.mean(h, axis=1, keepdims=True)
    var = jnp.mean((h - mu) ** 2, axis=1, keepdims=True)
    hn = (h - mu) * lax.rsqrt(var + 1e-5)
    y = (hn * g_ref[0] + bt_ref[0]).astype(jnp.bfloat16)
    cp.wait()
    y2 = lax.dot_general(
        y, w2_buf[...].astype(jnp.bfloat16), (((1,), (0,)), ((), ())),
        preferred_element_type=jnp.float32) + b2_ref[0]
    for b in range(B):
        sre = sr_ref[b, 0]
        wme = wm_ref[b, 0]
        pw = jnp.where(iota_k == sre, wme, 0.0).astype(jnp.bfloat16)
        y2b = lax.slice_in_dim(y2, b * TOPK, (b + 1) * TOPK, axis=0)
        contrib = lax.dot_general(
            pw, y2b.astype(jnp.bfloat16), (((0,), (0,)), ((), ())),
            preferred_element_type=jnp.float32)

        @pl.when(e == 0)
        def _():
            out_ref[b] = contrib

        @pl.when(e > 0)
        def _():
            out_ref[b] = out_ref[b] + contrib


@jax.jit
def kernel(x, Wg, bg, W1, b1, gamma, beta, W2, b2):
    f32 = jnp.float32
    bf16 = jnp.bfloat16
    gate = pl.pallas_call(
        _gate_kernel,
        grid=(B,),
        in_specs=[
            pl.BlockSpec((1, N, DIM), lambda b: (b, 0, 0)),
            pl.BlockSpec((DIM, NUM_EXPERTS), lambda b: (0, 0)),
            pl.BlockSpec((1, NUM_EXPERTS), lambda b: (0, 0)),
            pl.BlockSpec((NUM_EXPERTS, 1), lambda b: (0, 0)),
        ],
        out_specs=[
            pl.BlockSpec((1, N, NUM_EXPERTS), lambda b: (b, 0, 0)),
            pl.BlockSpec((1, NUM_EXPERTS, 1, N), lambda b: (b, 0, 0, 0)),
            pl.BlockSpec((1, NUM_EXPERTS, 1, N), lambda b: (b, 0, 0, 0)),
            pl.BlockSpec((1, N, DIM), lambda b: (b, 0, 0)),
        ],
        out_shape=[
            jax.ShapeDtypeStruct((B, N, NUM_EXPERTS), f32),
            jax.ShapeDtypeStruct((B, NUM_EXPERTS, 1, N), f32),
            jax.ShapeDtypeStruct((B, NUM_EXPERTS, 1, N), f32),
            jax.ShapeDtypeStruct((B, N, DIM), bf16),
        ],
    )
    x_gated, srank, wm, xbf = gate(
        x, Wg, bg.reshape(1, NUM_EXPERTS), bg.reshape(NUM_EXPERTS, 1))

    moe = pl.pallas_call(
        _ffnc_kernel,
        grid=(NUM_EXPERTS,),
        in_specs=[
            pl.BlockSpec((B, N, DIM), lambda e: (0, 0, 0)),
            pl.BlockSpec((B, 1, 1, N), lambda e: (0, e, 0, 0)),
            pl.BlockSpec((B, 1, 1, N), lambda e: (0, e, 0, 0)),
            pl.BlockSpec((1, DIM, FFN), lambda e: (e, 0, 0)),
            pl.BlockSpec((1, 1, FFN), lambda e: (e, 0, 0)),
            pl.BlockSpec((1, 1, FFN), lambda e: (e, 0, 0)),
            pl.BlockSpec((1, 1, FFN), lambda e: (e, 0, 0)),
            pl.BlockSpec(memory_space=pl.ANY),
            pl.BlockSpec((1, 1, OUT), lambda e: (e, 0, 0)),
        ],
        out_specs=pl.BlockSpec((B, N, OUT), lambda e: (0, 0, 0)),
        out_shape=jax.ShapeDtypeStruct((B, N, OUT), f32),
        scratch_shapes=[
            pltpu.VMEM((FFN, OUT), f32),
            pltpu.SemaphoreType.DMA,
        ],
    )(xbf, srank, wm, W1, b1.reshape(NUM_EXPERTS, 1, FFN),
      gamma.reshape(NUM_EXPERTS, 1, FFN), beta.reshape(NUM_EXPERTS, 1, FFN),
      W2, b2.reshape(NUM_EXPERTS, 1, OUT))
    return (moe, x_gated)


# speedup vs baseline: 1.2053x; 1.0555x over previous
import functools

import jax
import jax.numpy as jnp
from jax import lax
from jax.experimental import pallas as pl
from jax.experimental.pallas import tpu as pltpu
from jax.experimental.pallas import tpu_sc as plsc

DIM = 1024
NUM_EXPERTS = 8
FFN = 2048
OUT = 1024
B = 2
N = 2048
TOPK = 256


def _sortable_keys(v):
    b = lax.bitcast_convert_type(v, jnp.int32)
    return jnp.where(b < 0, jnp.bitwise_xor(jnp.invert(b), -2**31), b)


def _cumsum_lanes(x):
    s = x
    k = 1
    while k < N:
        s = s + jnp.concatenate(
            [jnp.zeros((s.shape[0], k), s.dtype), s[:, :N - k]], axis=1)
        k *= 2
    return s


def _gate_kernel(x_ref, wg_ref, bgr_ref, bgc_ref,
                 xg_ref, srank_ref, wm_ref, idx_ref):
    xb = x_ref[0]
    logits = lax.dot_general(
        xb, wg_ref[...], (((1,), (0,)), ((), ())),
        precision=lax.Precision.DEFAULT,
        preferred_element_type=jnp.float32) + bgr_ref[...]
    xg_ref[0] = logits
    lt = lax.dot_general(
        wg_ref[...], xb, (((0,), (1,)), ((), ())),
        precision=lax.Precision.DEFAULT,
        preferred_element_type=jnp.float32) + bgc_ref[...]

    keys = _sortable_keys(lt)
    lo = jnp.full((NUM_EXPERTS, 1), -2**31, jnp.int32)
    hi = jnp.full((NUM_EXPERTS, 1), 2**31 - 1, jnp.int32)
    for _ in range(33):
        mid = (lo >> 1) + (hi >> 1) + (jnp.bitwise_or(lo, hi) & 1)
        cnt = jnp.sum((keys >= mid).astype(jnp.int32), axis=1, keepdims=True)
        ok = cnt >= TOPK
        lo = jnp.where(ok, mid, lo)
        hi = jnp.where(ok, hi, mid - 1)
    thr = lo

    gt = keys > thr
    eq = keys == thr
    n_gt = jnp.sum(gt.astype(jnp.int32), axis=1, keepdims=True)
    needed = (TOPK - n_gt).astype(jnp.float32)
    cum_eq = _cumsum_lanes(eq.astype(jnp.float32))
    sel = jnp.logical_or(gt, jnp.logical_and(eq, cum_eq <= needed))
    rank = _cumsum_lanes(sel.astype(jnp.float32)) - 1.0
    srm = jnp.where(sel, rank, -1.0)
    srank_ref[0] = srm.reshape(NUM_EXPERTS, 1, N)

    ni = lax.broadcasted_iota(jnp.int32, (2, N), 1)
    qr = jnp.concatenate([
        lax.slice_in_dim(ni >> 4, 0, 1, axis=0),
        lax.slice_in_dim(ni & 15, 1, 2, axis=0)], axis=0).astype(jnp.bfloat16)
    iota_k = lax.broadcasted_iota(jnp.int32, (TOPK, N), 0).astype(jnp.float32)
    bb = pl.program_id(0)
    rows = []
    for e in range(NUM_EXPERTS):
        sre = lax.slice_in_dim(srm, e, e + 1, axis=0)
        pmat = (iota_k == sre).astype(jnp.bfloat16)
        qri = lax.dot_general(qr, pmat, (((1,), (1,)), ((), ())),
                              preferred_element_type=jnp.float32)
        rows.append(16.0 * lax.slice_in_dim(qri, 0, 1, axis=0)
                    + lax.slice_in_dim(qri, 1, 2, axis=0))
    idxf = jnp.concatenate(rows, axis=0)
    idx_ref[0] = idxf.astype(jnp.int32) + bb * N

    m = jnp.max(lt, axis=0, keepdims=True)
    p = jnp.exp(lt - m)
    w = p / jnp.sum(p, axis=0, keepdims=True)
    wm_ref[0] = jnp.where(sel, w, 0.0).reshape(NUM_EXPERTS, 1, N)


def _route_gather_kernel(idx_hbm, x_hbm, xe_hbm, idx_v, rows_v, sem):
    wid = lax.axis_index("s") * 2 + lax.axis_index("c")
    pair = wid // 2
    h = wid % 2
    b = pair % B
    e = pair // B
    row = b * NUM_EXPERTS + e
    for c in range(2):
        pltpu.sync_copy(idx_hbm.at[row, pl.ds(h * 128 + c * 64, 64)], idx_v)
        pltpu.async_copy(x_hbm.at[idx_v], rows_v, sem).wait()
        pltpu.sync_copy(rows_v, xe_hbm.at[wid, pl.ds(c * 64, 64)])


def _ffn_kernel(xe_ref, w1_ref, b1_ref, g_ref, bt_ref, w2_ref, b2_ref,
                y_ref):
    xe = xe_ref[0].reshape(B * TOPK, DIM).astype(jnp.bfloat16)
    h = lax.dot_general(
        xe, w1_ref[0].astype(jnp.bfloat16), (((1,), (0,)), ((), ())),
        preferred_element_type=jnp.float32) + b1_ref[0]
    h = 0.5 * h * (1.0 + lax.erf(h * 0.7071067811865476))
    mu = jnp.mean(h, axis=1, keepdims=True)
    var = jnp.mean((h - mu) ** 2, axis=1, keepdims=True)
    hn = (h - mu) * lax.rsqrt(var + 1e-5)
    y = hn * g_ref[0] + bt_ref[0]
    y2 = lax.dot_general(
        y.astype(jnp.bfloat16), w2_ref[0].astype(jnp.bfloat16),
        (((1,), (0,)), ((), ())),
        preferred_element_type=jnp.float32) + b2_ref[0]
    y_ref[0] = y2.astype(jnp.bfloat16).reshape(B, TOPK, OUT)


def _combine_kernel(y_ref, srank_ref, wm_ref, out_ref):
    iota_k = lax.broadcasted_iota(jnp.int32, (TOPK, N), 0).astype(jnp.float32)
    acc = jnp.zeros((N, OUT), jnp.float32)
    for e in range(NUM_EXPERTS):
        sre = srank_ref[0, e]
        wme = wm_ref[0, e]
        pw = jnp.where(iota_k == sre, wme, 0.0).astype(jnp.bfloat16)
        acc = acc + lax.dot_general(
            pw, y_ref[e, 0], (((0,), (0,)), ((), ())),
            preferred_element_type=jnp.float32)
    out_ref[0] = acc


@jax.jit
def kernel(x, Wg, bg, W1, b1, gamma, beta, W2, b2):
    f32 = jnp.float32
    bf16 = jnp.bfloat16
    gate = pl.pallas_call(
        _gate_kernel,
        grid=(B,),
        in_specs=[
            pl.BlockSpec((1, N, DIM), lambda b: (b, 0, 0)),
            pl.BlockSpec((DIM, NUM_EXPERTS), lambda b: (0, 0)),
            pl.BlockSpec((1, NUM_EXPERTS), lambda b: (0, 0)),
            pl.BlockSpec((NUM_EXPERTS, 1), lambda b: (0, 0)),
        ],
        out_specs=[
            pl.BlockSpec((1, N, NUM_EXPERTS), lambda b: (b, 0, 0)),
            pl.BlockSpec((1, NUM_EXPERTS, 1, N), lambda b: (b, 0, 0, 0)),
            pl.BlockSpec((1, NUM_EXPERTS, 1, N), lambda b: (b, 0, 0, 0)),
            pl.BlockSpec((1, NUM_EXPERTS, TOPK), lambda b: (b, 0, 0)),
        ],
        out_shape=[
            jax.ShapeDtypeStruct((B, N, NUM_EXPERTS), f32),
            jax.ShapeDtypeStruct((B, NUM_EXPERTS, 1, N), f32),
            jax.ShapeDtypeStruct((B, NUM_EXPERTS, 1, N), f32),
            jax.ShapeDtypeStruct((B, NUM_EXPERTS, TOPK), jnp.int32),
        ],
    )
    x_gated, srank, wm, idxg = gate(
        x, Wg, bg.reshape(1, NUM_EXPERTS), bg.reshape(NUM_EXPERTS, 1))

    route_gather = functools.partial(
        pl.kernel,
        out_type=jax.ShapeDtypeStruct((2 * NUM_EXPERTS * B, 128, DIM), f32),
        mesh=plsc.VectorSubcoreMesh(core_axis_name="c", subcore_axis_name="s"),
        scratch_types=[
            pltpu.VMEM((64,), jnp.int32),
            pltpu.VMEM((64, DIM), jnp.float32),
            pltpu.SemaphoreType.DMA,
        ],
    )(_route_gather_kernel)
    xe32 = route_gather(idxg.reshape(B * NUM_EXPERTS, TOPK),
                        x.reshape(B * N, DIM))
    xe = xe32.reshape(NUM_EXPERTS, B, TOPK, DIM)

    yslots = pl.pallas_call(
        _ffn_kernel,
        grid=(NUM_EXPERTS,),
        in_specs=[
            pl.BlockSpec((1, B, TOPK, DIM), lambda e: (e, 0, 0, 0)),
            pl.BlockSpec((1, DIM, FFN), lambda e: (e, 0, 0)),
            pl.BlockSpec((1, 1, FFN), lambda e: (e, 0, 0)),
            pl.BlockSpec((1, 1, FFN), lambda e: (e, 0, 0)),
            pl.BlockSpec((1, 1, FFN), lambda e: (e, 0, 0)),
            pl.BlockSpec((1, FFN, OUT), lambda e: (e, 0, 0)),
            pl.BlockSpec((1, 1, OUT), lambda e: (e, 0, 0)),
        ],
        out_specs=pl.BlockSpec((1, B, TOPK, OUT), lambda e: (e, 0, 0, 0)),
        out_shape=jax.ShapeDtypeStruct((NUM_EXPERTS, B, TOPK, OUT), bf16),
    )(xe, W1, b1.reshape(NUM_EXPERTS, 1, FFN),
      gamma.reshape(NUM_EXPERTS, 1, FFN), beta.reshape(NUM_EXPERTS, 1, FFN),
      W2, b2.reshape(NUM_EXPERTS, 1, OUT))

    moe = pl.pallas_call(
        _combine_kernel,
        grid=(B,),
        in_specs=[
            pl.BlockSpec((NUM_EXPERTS, 1, TOPK, OUT), lambda b: (0, b, 0, 0)),
            pl.BlockSpec((1, NUM_EXPERTS, 1, N), lambda b: (b, 0, 0, 0)),
            pl.BlockSpec((1, NUM_EXPERTS, 1, N), lambda b: (b, 0, 0, 0)),
        ],
        out_specs=pl.BlockSpec((1, N, OUT), lambda b: (b, 0, 0)),
        out_shape=jax.ShapeDtypeStruct((B, N, OUT), f32),
    )(yslots, srank, wm)
    return (moe, x_gated)
